# baseline (speedup 1.0000x reference)
import functools

import jax
import jax.numpy as jnp
from jax import lax
from jax.experimental import pallas as pl
from jax.experimental.pallas import tpu as pltpu
from jax.experimental.pallas import tpu_sc as plsc

N = 10000
E = 320000
D = 128
G = 256
NC = 2
NS = 16
NW = NC * NS

NPAD = 10240
ROWS_PER_TILE = NPAD // NW
ROWS_PER_SUB = NPAD // NS
EC = 10240
CH = 64
NCH = EC // CH
NCHQ = NCH // 4
GPAD = 384
DUMMY = N


_MESH = plsc.VectorSubcoreMesh(core_axis_name="c", subcore_axis_name="s")


ACC_H = 6144
ACC_OFF = 4096
ACC_PT = ACC_H // NS
STRIP = (NPAD - ACC_H) // NS


def _scale_rows(rows, ns, nd):
    def srow(e, carry):
        nv = ns[e, pl.ds(0, 16)] * nd[e, pl.ds(0, 16)]
        for f in range(D // 16):
            sl = pl.ds(f * 16, 16)
            rows[e, sl] = rows[e, sl] * nv
        return carry

    lax.fori_loop(0, CH, srow, 0)


def _hop_body(h_hbm, dv_hbm, srcp, dstp, dstl, z_hbm, out_hbm,
              sidx, didx, didxl, rows0, rows1, ns0, nd0, ns1, nd1,
              acc, sem0, sem1):
    c = lax.axis_index("c")
    s = lax.axis_index("s")
    w = c * NS + s
    pltpu.sync_copy(z_hbm.at[pl.ds(0, ACC_PT)],
                    acc.at[pl.ds(s * ACC_PT, ACC_PT)])
    strip0 = jnp.where(c == 0, ACC_H, 0) + s * STRIP
    pltpu.sync_copy(z_hbm.at[pl.ds(0, STRIP)],
                    out_hbm.at[c, pl.ds(strip0, STRIP)])
    plsc.subcore_barrier()

    def gather(j, rows, ns, nd, sem):
        pltpu.async_copy(h_hbm.at[sidx.at[j]], rows, sem)
        pltpu.async_copy(dv_hbm.at[sidx.at[j]], ns, sem)
        pltpu.async_copy(dv_hbm.at[didx.at[j]], nd, sem)

    def wait(j, rows, ns, nd, sem):
        pltpu.make_async_copy(h_hbm.at[sidx.at[j]], rows, sem).wait()
        pltpu.make_async_copy(dv_hbm.at[sidx.at[j]], ns, sem).wait()
        pltpu.make_async_copy(dv_hbm.at[didx.at[j]], nd, sem).wait()

    def body(j2, carry):
        j = j2 * 2
        gather(j + 1, rows1, ns1, nd1, sem1)
        wait(j, rows0, ns0, nd0, sem0)
        _scale_rows(rows0, ns0, nd0)
        pltpu.sync_copy(rows0, acc.at[didxl.at[j]], add=True)

        @pl.when(j2 < NCHQ // 2 - 1)
        def _():
            gather(j + 2, rows0, ns0, nd0, sem0)

        wait(j + 1, rows1, ns1, nd1, sem1)
        _scale_rows(rows1, ns1, nd1)
        pltpu.sync_copy(rows1, acc.at[didxl.at[j + 1]], add=True)
        return carry

    for q in range(4):
        pltpu.sync_copy(srcp.at[w, pl.ds(q * NCHQ, NCHQ)], sidx)
        pltpu.sync_copy(dstp.at[w, pl.ds(q * NCHQ, NCHQ)], didx)
        pltpu.sync_copy(dstl.at[w, pl.ds(q * NCHQ, NCHQ)], didxl)
        gather(0, rows0, ns0, nd0, sem0)
        lax.fori_loop(0, NCHQ // 2, body, 0)

    plsc.subcore_barrier()
    out0 = jnp.where(c == 0, 0, ACC_OFF) + s * ACC_PT
    pltpu.sync_copy(acc.at[pl.ds(s * ACC_PT, ACC_PT)],
                    out_hbm.at[c, pl.ds(out0, ACC_PT)])


_hop = functools.partial(
    pl.kernel,
    _hop_body,
    out_type=jax.ShapeDtypeStruct((NC, NPAD, D), jnp.float32),
    mesh=_MESH,
    scratch_types=[
        pltpu.VMEM((NCHQ, CH), jnp.int32),
        pltpu.VMEM((NCHQ, CH), jnp.int32),
        pltpu.VMEM((NCHQ, CH), jnp.int32),
        pltpu.VMEM((CH, D), jnp.float32),
        pltpu.VMEM((CH, D), jnp.float32),
        pltpu.VMEM((CH, D), jnp.float32),
        pltpu.VMEM((CH, D), jnp.float32),
        pltpu.VMEM((CH, D), jnp.float32),
        pltpu.VMEM((CH, D), jnp.float32),
        pltpu.VMEM_SHARED((ACC_H, D), jnp.float32),
        pltpu.SemaphoreType.DMA,
        pltpu.SemaphoreType.DMA,
    ],
)()


def _prep_body(dstp, batchp, ones_hbm, z_hbm, deg_out, cnt_out,
               didx, bidx, ones_v, dacc, cacc):
    c = lax.axis_index("c")
    s = lax.axis_index("s")
    w = c * NS + s
    pltpu.sync_copy(batchp.at[w], bidx)
    pltpu.sync_copy(ones_hbm, ones_v)
    pltpu.sync_copy(z_hbm, dacc.at[pl.ds(s * ROWS_PER_SUB, ROWS_PER_SUB)])

    @pl.when(s == 0)
    def _():
        pltpu.sync_copy(z_hbm.at[pl.ds(0, GPAD)], cacc)

    plsc.subcore_barrier()

    def body(j, carry):
        pltpu.sync_copy(ones_v, dacc.at[didx.at[j]], add=True)
        return carry

    for q in range(4):
        pltpu.sync_copy(dstp.at[w, pl.ds(q * NCHQ, NCHQ)], didx)
        lax.fori_loop(0, NCHQ, body, 0)

    def body2(k, carry):
        pltpu.sync_copy(ones_v.at[pl.ds(0, 64)], cacc.at[bidx.at[k]], add=True)
        return carry

    lax.fori_loop(0, ROWS_PER_TILE // 64, body2, 0)
    plsc.subcore_barrier()
    pltpu.sync_copy(dacc.at[pl.ds(s * ROWS_PER_SUB, ROWS_PER_SUB)],
                    deg_out.at[c, pl.ds(s * ROWS_PER_SUB, ROWS_PER_SUB)])
    pltpu.sync_copy(cacc.at[pl.ds(s * (GPAD // NS), GPAD // NS)],
                    cnt_out.at[c, pl.ds(s * (GPAD // NS), GPAD // NS)])


_prep = functools.partial(
    pl.kernel,
    _prep_body,
    out_type=(jax.ShapeDtypeStruct((NC, NPAD, D), jnp.float32),
              jax.ShapeDtypeStruct((NC, GPAD, D), jnp.float32)),
    mesh=_MESH,
    scratch_types=[
        pltpu.VMEM((NCHQ, CH), jnp.int32),
        pltpu.VMEM((ROWS_PER_TILE // 64, 64), jnp.int32),
        pltpu.VMEM((CH, D), jnp.float32),
        pltpu.VMEM_SHARED((NPAD, D), jnp.float32),
        pltpu.VMEM_SHARED((GPAD, D), jnp.float32),
    ],
)()


_PQ = (NPAD // 64) // 4


def _pool_body(h_hbm, batchp2, z_hbm, out_hbm, bidx, rows, pacc):
    c = lax.axis_index("c")
    s = lax.axis_index("s")

    @pl.when(jnp.logical_and(c == 0, s == 0))
    def _():
        pltpu.sync_copy(z_hbm.at[pl.ds(0, GPAD)], pacc)
        for q in range(4):
            pltpu.sync_copy(batchp2.at[pl.ds(q * _PQ, _PQ)], bidx)

            def body(k, carry):
                pltpu.sync_copy(h_hbm.at[pl.ds((q * _PQ + k) * 64, 64)], rows)
                pltpu.sync_copy(rows, pacc.at[bidx.at[k]], add=True)
                return carry

            lax.fori_loop(0, _PQ, body, 0)
        pltpu.sync_copy(pacc, out_hbm)


_pool = functools.partial(
    pl.kernel,
    _pool_body,
    out_type=jax.ShapeDtypeStruct((GPAD, D), jnp.float32),
    mesh=_MESH,
    scratch_types=[
        pltpu.VMEM((_PQ, 64), jnp.int32),
        pltpu.VMEM((64, D), jnp.float32),
        pltpu.VMEM_SHARED((GPAD, D), jnp.float32),
    ],
)()


_BN = 1024
_NBLK = NPAD // _BN


def _dinv_body(da_ref, db_ref, out_ref):
    i = pl.program_id(0)
    deg = da_ref[:, :1] + db_ref[:, :1] + 1.0
    row = lax.broadcasted_iota(jnp.int32, (_BN, 1), 0) + i * _BN
    dv = jnp.where(row < N, 1.0 / jnp.sqrt(deg), 0.0)
    out_ref[...] = jnp.broadcast_to(dv, (_BN, D))


def _dinv_tc(da, db):
    return pl.pallas_call(
        _dinv_body,
        grid=(_NBLK,),
        in_specs=[pl.BlockSpec((_BN, D), lambda i: (i, 0)),
                  pl.BlockSpec((_BN, D), lambda i: (i, 0))],
        out_specs=pl.BlockSpec((_BN, D), lambda i: (i, 0)),
        out_shape=jax.ShapeDtypeStruct((NPAD, D), jnp.float32),
    )(da, db)


def _embed_body(x_ref, w_ref, b_ref, out_ref):
    h = jnp.dot(x_ref[...], w_ref[...], preferred_element_type=jnp.float32)
    out_ref[...] = h + b_ref[...]


def _embed_tc(x, w, b):
    return pl.pallas_call(
        _embed_body,
        grid=(_NBLK,),
        in_specs=[pl.BlockSpec((_BN, D), lambda i: (i, 0)),
                  pl.BlockSpec((D, D), lambda i: (0, 0)),
                  pl.BlockSpec((1, D), lambda i: (0, 0))],
        out_specs=pl.BlockSpec((_BN, D), lambda i: (i, 0)),
        out_shape=jax.ShapeDtypeStruct((NPAD, D), jnp.float32),
    )(x, w, b)


def _comb_body(a_ref, b_ref, h_ref, dv_ref, out_ref):
    dv = dv_ref[...]
    out_ref[...] = a_ref[...] + b_ref[...] + (dv * dv) * h_ref[...]


def _comb_tc(a, b, h, dv):
    return pl.pallas_call(
        _comb_body,
        grid=(_NBLK,),
        in_specs=[pl.BlockSpec((_BN, D), lambda i: (i, 0))] * 4,
        out_specs=pl.BlockSpec((_BN, D), lambda i: (i, 0)),
        out_shape=jax.ShapeDtypeStruct((NPAD, D), jnp.float32),
    )(a, b, h, dv)


def _comb_mm_body(relu, a_ref, b_ref, h_ref, dv_ref, w_ref, bias_ref, out_ref):
    dv = dv_ref[...]
    t = a_ref[...] + b_ref[...] + (dv * dv) * h_ref[...]
    r = jnp.dot(t, w_ref[...], preferred_element_type=jnp.float32) + bias_ref[...]
    if relu:
        r = jnp.maximum(r, 0.0)
    out_ref[...] = r


def _comb_mm_tc(a, b, h, dv, w, bias, relu):
    return pl.pallas_call(
        functools.partial(_comb_mm_body, relu),
        grid=(_NBLK,),
        in_specs=[pl.BlockSpec((_BN, D), lambda i: (i, 0)),
                  pl.BlockSpec((_BN, D), lambda i: (i, 0)),
                  pl.BlockSpec((_BN, D), lambda i: (i, 0)),
                  pl.BlockSpec((_BN, D), lambda i: (i, 0)),
                  pl.BlockSpec((D, D), lambda i: (0, 0)),
                  pl.BlockSpec((1, D), lambda i: (0, 0))],
        out_specs=pl.BlockSpec((_BN, D), lambda i: (i, 0)),
        out_shape=jax.ShapeDtypeStruct((NPAD, D), jnp.float32),
    )(a, b, h, dv, w, bias)


def _task_body(p_ref, ca_ref, cb_ref, w1_ref, b1_ref, gm_ref,
               bt_ref, w2_ref, b2_ref, out_ref):
    cnt = ca_ref[:, :1] + cb_ref[:, :1]
    pooled = p_ref[...] / jnp.maximum(cnt, 1.0)
    z = jnp.dot(pooled, w1_ref[...], preferred_element_type=jnp.float32) + b1_ref[...]
    mu = jnp.mean(z, axis=0, keepdims=True)
    var = jnp.mean((z - mu) ** 2, axis=0, keepdims=True)
    z = (z - mu) / jnp.sqrt(var + 1e-5) * gm_ref[...] + bt_ref[...]
    z = jnp.maximum(z, 0.0)
    out_ref[...] = jnp.dot(z, w2_ref[...], preferred_element_type=jnp.float32) + b2_ref[...]


def _task_tc(p, ca, cb, w1, b1, gm, bt, w2, b2):
    return pl.pallas_call(
        _task_body,
        out_shape=jax.ShapeDtypeStruct((G, D), jnp.float32),
    )(p, ca, cb, w1, b1, gm, bt, w2, b2)


def kernel(x, edge_index, batch, W_embed, b_embed, W1, b1, W2, b2,
           Wt1, bt1, gamma, beta, Wt2, bt2):
    f32 = jnp.float32
    i32 = jnp.int32

    xp = jnp.pad(x, ((0, NPAD - N), (0, 0)))
    src = edge_index[0].astype(i32)
    dst = edge_index[1].astype(i32)
    perm = jnp.argsort(dst, stable=True)
    srcs = src[perm]
    dsts = dst[perm]
    tgt = jnp.arange(NW, dtype=i32) * (E // NW)
    starts = jnp.searchsorted(dsts, dsts[tgt], side="left").astype(i32)
    starts = starts.at[0].set(0)
    bounds = jnp.concatenate([starts, jnp.array([E], i32)])
    eidx = jnp.arange(E, dtype=i32)
    tile = jnp.searchsorted(bounds, eidx, side="right").astype(i32) - 1
    flat = tile * EC + (eidx - bounds[tile])
    srcp = jnp.full((NW * EC,), DUMMY, i32).at[flat].set(srcs).reshape(NW, NCH, CH)
    dstp = jnp.full((NW * EC,), DUMMY, i32).at[flat].set(dsts).reshape(NW, NCH, CH)
    off = jnp.where(jnp.arange(NW) >= NS, ACC_OFF, 0)[:, None, None]
    dstl = jnp.where(dstp == DUMMY, off, dstp) - off
    batchp = jnp.concatenate(
        [batch.astype(i32), jnp.full((NPAD - N,), G, i32)]
    ).reshape(NW, ROWS_PER_TILE // 64, 64)
    batchp2 = batchp.reshape(NPAD // 64, 64)

    z640 = jnp.zeros((ROWS_PER_SUB, D), f32)
    ones_rows = jnp.ones((CH, D), f32)

    b_embed2 = b_embed.reshape(1, D)
    b1_2 = b1.reshape(1, D)
    b2_2 = b2.reshape(1, D)
    bt1_2 = bt1.reshape(1, D)
    gamma2 = gamma.reshape(1, D)
    beta2 = beta.reshape(1, D)
    Wt2p = jnp.pad(Wt2, ((0, 0), (0, D - Wt2.shape[1])))
    bt2p = jnp.pad(bt2, (0, D - bt2.shape[0])).reshape(1, D)

    deg2, cnt2 = _prep(dstp, batchp, ones_rows, z640)
    dinv_b = _dinv_tc(deg2[0], deg2[1])

    h0 = _embed_tc(xp, W_embed, b_embed2)
    o = _hop(h0, dinv_b, srcp, dstp, dstl, z640)
    h1 = _comb_tc(o[0], o[1], h0, dinv_b)
    o = _hop(h1, dinv_b, srcp, dstp, dstl, z640)
    h3 = _comb_mm_tc(o[0], o[1], h1, dinv_b, W1, b1_2, relu=True)
    o = _hop(h3, dinv_b, srcp, dstp, dstl, z640)
    h4 = _comb_tc(o[0], o[1], h3, dinv_b)
    o = _hop(h4, dinv_b, srcp, dstp, dstl, z640)
    h6 = _comb_mm_tc(o[0], o[1], h4, dinv_b, W2, b2_2, relu=False)

    po = _pool(h6, batchp2, z640)
    zt = _task_tc(po[:G], cnt2[0, :G], cnt2[1, :G],
                  Wt1, bt1_2, gamma2, beta2, Wt2p, bt2p)
    return zt[:, :1]
